# baseline (speedup 1.0000x reference)
import jax
import jax.numpy as jnp
from jax.experimental import pallas as pl
from jax.experimental.pallas import tpu as pltpu
from jax.experimental.pallas import tpu_sc as plsc

_B = 512
_R = 4 * 64 * 64
_RB = 2048

_NCORES = 2
_NSUB = 16
_PER = _B // (_NCORES * _NSUB)


def _sc_gather(t, sa, som):
    mesh = plsc.VectorSubcoreMesh(core_axis_name="c", subcore_axis_name="s")

    @pl.kernel(
        out_type=(
            jax.ShapeDtypeStruct((1, _B), jnp.float32),
            jax.ShapeDtypeStruct((1, _B), jnp.float32),
        ),
        mesh=mesh,
        scratch_types=[
            pltpu.VMEM((1, _PER), jnp.int32),
            pltpu.VMEM((1, _PER), jnp.float32),
            pltpu.VMEM((1, _PER), jnp.float32),
        ],
    )
    def gather_kernel(t_hbm, sa_hbm, som_hbm, a_hbm, b_hbm, tv, av, bv):
        c = jax.lax.axis_index("c")
        s = jax.lax.axis_index("s")
        base = (c * _NSUB + s) * _PER
        sl = pl.ds(base, _PER)
        pltpu.sync_copy(t_hbm.at[0, sl], tv.at[0])
        pltpu.sync_copy(sa_hbm.at[tv.at[0]], av.at[0])
        pltpu.sync_copy(som_hbm.at[tv.at[0]], bv.at[0])
        pltpu.sync_copy(av.at[0], a_hbm.at[0, sl])
        pltpu.sync_copy(bv.at[0], b_hbm.at[0, sl])

    return gather_kernel(t.reshape(1, _B).astype(jnp.int32), sa, som)


def _dense_body(a_ref, b_ref, x_ref, n_ref, o_ref):
    o_ref[...] = a_ref[...] * x_ref[...] + b_ref[...] * n_ref[...]


def kernel(x0, t, noise, sqrt_alphas_cumprod, sqrt_one_minus_alphas_cumprod):
    a_row, b_row = _sc_gather(
        t, sqrt_alphas_cumprod, sqrt_one_minus_alphas_cumprod
    )
    xT = jnp.transpose(x0, (1, 2, 3, 0)).reshape(_R, _B)
    nT = jnp.transpose(noise, (1, 2, 3, 0)).reshape(_R, _B)
    out = pl.pallas_call(
        _dense_body,
        grid=(_R // _RB,),
        in_specs=[
            pl.BlockSpec((1, _B), lambda i: (0, 0)),
            pl.BlockSpec((1, _B), lambda i: (0, 0)),
            pl.BlockSpec((_RB, _B), lambda i: (i, 0)),
            pl.BlockSpec((_RB, _B), lambda i: (i, 0)),
        ],
        out_specs=pl.BlockSpec((_RB, _B), lambda i: (i, 0)),
        out_shape=jax.ShapeDtypeStruct((_R, _B), jnp.float32),
        compiler_params=pltpu.CompilerParams(
            dimension_semantics=("parallel",),
        ),
    )(a_row, b_row, xT, nT)
    return out.reshape(4, 64, 64, _B).transpose(3, 0, 1, 2)


# speedup vs baseline: 1.0108x; 1.0108x over previous
import jax
import jax.numpy as jnp
from jax.experimental import pallas as pl
from jax.experimental.pallas import tpu as pltpu
from jax.experimental.pallas import tpu_sc as plsc

_B = 512
_R = 4 * 64 * 64
_RB = 2048

_NCORES = 2
_NSUB = 16
_PER = _B // (_NCORES * _NSUB)


def _sc_gather(t, sa, som):
    mesh = plsc.VectorSubcoreMesh(core_axis_name="c", subcore_axis_name="s")

    @pl.kernel(
        out_type=(
            jax.ShapeDtypeStruct((1, _B), jnp.float32),
            jax.ShapeDtypeStruct((1, _B), jnp.float32),
        ),
        mesh=mesh,
        scratch_types=[
            pltpu.VMEM((1, _PER), jnp.int32),
            pltpu.VMEM((1, _PER), jnp.float32),
            pltpu.VMEM((1, _PER), jnp.float32),
            pltpu.SemaphoreType.DMA,
            pltpu.SemaphoreType.DMA,
        ],
    )
    def gather_kernel(t_hbm, sa_hbm, som_hbm, a_hbm, b_hbm, tv, av, bv,
                      sem0, sem1):
        c = jax.lax.axis_index("c")
        s = jax.lax.axis_index("s")
        base = (c * _NSUB + s) * _PER
        sl = pl.ds(base, _PER)
        pltpu.async_copy(t_hbm.at[0, sl], tv.at[0], sem0).wait()
        cpa = pltpu.async_copy(sa_hbm.at[tv.at[0]], av.at[0], sem0)
        cpb = pltpu.async_copy(som_hbm.at[tv.at[0]], bv.at[0], sem1)
        cpa.wait()
        cpb.wait()
        cpo1 = pltpu.async_copy(av.at[0], a_hbm.at[0, sl], sem0)
        cpo2 = pltpu.async_copy(bv.at[0], b_hbm.at[0, sl], sem1)
        cpo1.wait()
        cpo2.wait()

    return gather_kernel(t.reshape(1, _B).astype(jnp.int32), sa, som)


def _dense_body(a_ref, b_ref, x_ref, n_ref, o_ref):
    o_ref[...] = a_ref[...] * x_ref[...] + b_ref[...] * n_ref[...]


def kernel(x0, t, noise, sqrt_alphas_cumprod, sqrt_one_minus_alphas_cumprod):
    a_row, b_row = _sc_gather(
        t, sqrt_alphas_cumprod, sqrt_one_minus_alphas_cumprod
    )
    xT = jnp.transpose(x0, (1, 2, 3, 0)).reshape(_R, _B)
    nT = jnp.transpose(noise, (1, 2, 3, 0)).reshape(_R, _B)
    out = pl.pallas_call(
        _dense_body,
        grid=(_R // _RB,),
        in_specs=[
            pl.BlockSpec((1, _B), lambda i: (0, 0)),
            pl.BlockSpec((1, _B), lambda i: (0, 0)),
            pl.BlockSpec((_RB, _B), lambda i: (i, 0)),
            pl.BlockSpec((_RB, _B), lambda i: (i, 0)),
        ],
        out_specs=pl.BlockSpec((_RB, _B), lambda i: (i, 0)),
        out_shape=jax.ShapeDtypeStruct((_R, _B), jnp.float32),
        compiler_params=pltpu.CompilerParams(
            dimension_semantics=("parallel",),
        ),
    )(a_row, b_row, xT, nT)
    return out.reshape(4, 64, 64, _B).transpose(3, 0, 1, 2)


# speedup vs baseline: 1.6028x; 1.5857x over previous
import jax
import jax.numpy as jnp
from jax.experimental import pallas as pl
from jax.experimental.pallas import tpu as pltpu

_B = 512
_R = 4 * 64 * 64
_RB = 2048
_T = 1000


def _body(t_ref, sa_ref, som_ref, x_ref, n_ref, o_ref, a_s, b_s):
    i = pl.program_id(0)

    @pl.when(i == 0)
    def _gather():
        t_row = t_ref[...]
        iota_j = jax.lax.broadcasted_iota(jnp.int32, (_T, _B), 0)
        oh = (iota_j == t_row).astype(jnp.float32)
        a_s[...] = jax.lax.dot_general(
            sa_ref[...], oh, (((1,), (0,)), ((), ())),
            preferred_element_type=jnp.float32,
        )
        b_s[...] = jax.lax.dot_general(
            som_ref[...], oh, (((1,), (0,)), ((), ())),
            preferred_element_type=jnp.float32,
        )

    o_ref[...] = a_s[...] * x_ref[...] + b_s[...] * n_ref[...]


def kernel(x0, t, noise, sqrt_alphas_cumprod, sqrt_one_minus_alphas_cumprod):
    xT = jnp.transpose(x0, (1, 2, 3, 0)).reshape(_R, _B)
    nT = jnp.transpose(noise, (1, 2, 3, 0)).reshape(_R, _B)
    t_row = t.astype(jnp.int32).reshape(1, _B)
    sa_row = sqrt_alphas_cumprod.reshape(1, _T)
    som_row = sqrt_one_minus_alphas_cumprod.reshape(1, _T)
    out = pl.pallas_call(
        _body,
        grid=(_R // _RB,),
        in_specs=[
            pl.BlockSpec((1, _B), lambda i: (0, 0)),
            pl.BlockSpec((1, _T), lambda i: (0, 0)),
            pl.BlockSpec((1, _T), lambda i: (0, 0)),
            pl.BlockSpec((_RB, _B), lambda i: (i, 0)),
            pl.BlockSpec((_RB, _B), lambda i: (i, 0)),
        ],
        out_specs=pl.BlockSpec((_RB, _B), lambda i: (i, 0)),
        out_shape=jax.ShapeDtypeStruct((_R, _B), jnp.float32),
        scratch_shapes=[
            pltpu.VMEM((1, _B), jnp.float32),
            pltpu.VMEM((1, _B), jnp.float32),
        ],
        compiler_params=pltpu.CompilerParams(
            dimension_semantics=("arbitrary",),
        ),
    )(t_row, sa_row, som_row, xT, nT)
    return out.reshape(4, 64, 64, _B).transpose(3, 0, 1, 2)
